# baseline (speedup 1.0000x reference)
import functools

import jax
import jax.numpy as jnp
from jax import lax
from jax.experimental import pallas as pl
from jax.experimental.pallas import tpu as pltpu
from jax.experimental.pallas import tpu_sc as plsc

BATCH = 4
SEQ = 2048
D = 1024
NC = 2
NS = 16
NW = NC * NS
L = 16

POS_PER_W = SEQ // NW
CHUNK = 32
N_HALF = POS_PER_W // CHUNK
NCHUNK = N_HALF * BATCH
VECS = CHUNK * (D // L)

_mesh = plsc.VectorSubcoreMesh(core_axis_name="c", subcore_axis_name="s")


@functools.partial(
    pl.kernel,
    mesh=_mesh,
    out_type=jax.ShapeDtypeStruct((BATCH, SEQ, D), jnp.float32),
    scratch_types=[
        pltpu.VMEM((BATCH, POS_PER_W), jnp.int32),
        pltpu.VMEM((CHUNK, D), jnp.float32),
        pltpu.VMEM((CHUNK, D), jnp.float32),
        pltpu.VMEM((CHUNK, D), jnp.float32),
        pltpu.SemaphoreType.DMA,
        pltpu.SemaphoreType.DMA,
        pltpu.SemaphoreType.DMA,
    ],
)
def _embed(ids_hbm, wte_hbm, wpe_hbm, out_hbm, ids_v, rows_a, rows_b, wpe_v,
           sem_g, sem_s, sem_w):
    wid = lax.axis_index("s") * NC + lax.axis_index("c")
    p0 = wid * POS_PER_W

    id_copies = [
        pltpu.async_copy(ids_hbm.at[b, pl.ds(p0, POS_PER_W)], ids_v.at[b],
                         sem_w)
        for b in range(BATCH)
    ]
    for c in id_copies:
        c.wait()

    rows = [rows_a, rows_b]

    def start_wpe(h):
        return pltpu.async_copy(
            wpe_hbm.at[pl.ds(p0 + h * CHUNK, CHUNK)], wpe_v, sem_w)

    def chunk_coords(t):
        h, b = divmod(t, BATCH)
        return h, b

    def start_gather(t):
        h, b = chunk_coords(t)
        return pltpu.async_copy(
            wte_hbm.at[ids_v.at[b, pl.ds(h * CHUNK, CHUNK)]],
            rows[t % 2], sem_g)

    def start_store(t):
        h, b = chunk_coords(t)
        return pltpu.async_copy(
            rows[t % 2], out_hbm.at[b, pl.ds(p0 + h * CHUNK, CHUNK)], sem_s)

    gathers = [None] * NCHUNK
    stores = [None] * NCHUNK

    gathers[0] = start_gather(0)
    wpe_pending = start_wpe(0)
    for t in range(NCHUNK):
        if t + 1 < NCHUNK:
            if t - 1 >= 0:
                stores[t - 1].wait()
            gathers[t + 1] = start_gather(t + 1)
        gathers[t].wait()
        h, b = chunk_coords(t)
        if b == 0:
            wpe_pending.wait()
        buf = rows[t % 2]

        def add_body(k, carry):
            i = k >> 6
            j = pl.multiple_of((k & 63) << 4, L)
            plsc.addupdate(buf.at[i, pl.ds(j, L)], wpe_v[i, pl.ds(j, L)])
            return carry

        lax.fori_loop(0, VECS, add_body, 0, unroll=8)
        if b == BATCH - 1 and h + 1 < N_HALF:
            wpe_pending = start_wpe(h + 1)
        stores[t] = start_store(t)
    stores[NCHUNK - 2].wait()
    stores[NCHUNK - 1].wait()


def kernel(input_ids, wte, wpe):
    return _embed(input_ids.astype(jnp.int32), wte, wpe)
